# baseline (speedup 1.0000x reference)
import functools
import math

import jax
import jax.numpy as jnp
from jax import lax
from jax.experimental import pallas as pl
from jax.experimental.pallas import tpu as pltpu
from jax.experimental.pallas import tpu_sc as plsc

HIDDEN = 64
SCALE = math.sqrt(HIDDEN)

NC = 2
NS = 16
NW = NC * NS

NTOK = 16384
SEQ = 50
VOCAB = 1000000
HALF = VOCAB // 2
IBS = 128
NIB = NTOK // IBS
ITEMS = SEQ * NIB
IPW = ITEMS // NW

_mesh = plsc.VectorSubcoreMesh(core_axis_name="c", subcore_axis_name="s")


@functools.partial(
    pl.kernel,
    mesh=_mesh,
    out_type=jax.ShapeDtypeStruct((SEQ, HIDDEN, NTOK), jnp.float32),
    scratch_types=[
        pltpu.VMEM((3, 8, IBS), jnp.int32),
        pltpu.VMEM((3, IBS), jnp.int32),
        pltpu.VMEM((3, IBS), jnp.int32),
        pltpu.VMEM((3, IBS, 128), jnp.float32),
        pltpu.VMEM((2, HIDDEN, IBS), jnp.float32),
        pltpu.SemaphoreType.DMA,
        pltpu.SemaphoreType.DMA,
        pltpu.SemaphoreType.DMA,
    ],
    compiler_params=pltpu.CompilerParams(
        use_tc_tiling_on_sc=True, needs_layout_passes=False
    ),
)
def _emb_lookup(
    xT_hbm, tp_hbm, out_hbm, idx_v, pidx_v, par_v, rows_v, outt_v,
    isem, gsem, ssem,
):
    wid = lax.axis_index("s") * NC + lax.axis_index("c")
    iota16 = lax.iota(jnp.int32, 16)
    diag = [(iota16 + k) & 15 for k in range(16)]
    base0 = wid * IPW

    def coords(m):
        item = base0 + m
        j = item // NIB
        ib = item - j * NIB
        return j, ib

    def idx_copy(m):
        j, ib = coords(m)
        return pltpu.make_async_copy(
            xT_hbm.at[pl.ds((j // 8) * 8, 8), pl.ds(ib * IBS, IBS)],
            idx_v.at[m % 3],
            isem,
        )

    def gather_copy(m):
        return pltpu.make_async_copy(
            tp_hbm.at[pidx_v.at[m % 3]], rows_v.at[m % 3], gsem
        )

    def store_copy(m):
        j, ib = coords(m)
        return pltpu.make_async_copy(
            outt_v.at[m & 1], out_hbm.at[j, :, pl.ds(ib * IBS, IBS)], ssem
        )

    def prep(m):
        buf = m % 3
        j, _ = coords(m)
        jr = j - (j // 8) * 8

        def pidx_body(k, _):
            sl = pl.ds(k * 16, 16)
            iv = idx_v[buf, jr, sl]
            pidx_v[buf, sl] = lax.shift_right_logical(iv, 1)
            par_v[buf, sl] = (iv & 1) * HIDDEN
            return 0

        lax.fori_loop(0, IBS // 16, pidx_body, 0)
        gather_copy(m).start()

    idx_copy(0).start()
    idx_copy(0).wait()
    prep(0)
    idx_copy(1).start()
    idx_copy(1).wait()
    prep(1)
    idx_copy(2).start()

    def item_body(n, _):
        buf = n % 3
        obuf = n & 1

        @pl.when(n + 2 < IPW)
        def _prep_next():
            idx_copy(n + 2).wait()
            prep(n + 2)

        @pl.when(n + 3 < IPW)
        def _stage_next3():
            idx_copy(n + 3).start()

        gather_copy(n).wait()

        @pl.when(n >= 2)
        def _drain():
            store_copy(n - 2).wait()

        rows = rows_v.at[buf]
        outt = outt_v.at[obuf]

        def grp_body(g, _):
            i0 = g * 16
            icol = i0 + iota16
            colbase = par_v[buf, pl.ds(i0, 16)]
            for hblk in range(0, HIDDEN, 16):
                cbh = colbase + hblk
                for k in range(16):
                    d = diag[k]
                    v = plsc.load_gather(rows, [icol, cbh + d])
                    plsc.store_scatter(outt, [hblk + d, icol], v * SCALE)
            return 0

        lax.fori_loop(0, IBS // 16, grp_body, 0)

        store_copy(n).start()
        return 0

    lax.fori_loop(0, IPW, item_body, 0)

    store_copy(IPW - 2).wait()
    store_copy(IPW - 1).wait()


def kernel(x, table):
    assert x.shape == (NTOK, SEQ) and table.shape == (VOCAB, HIDDEN)
    xT = x.astype(jnp.int32).T
    tp = table.reshape(HALF, 2 * HIDDEN)
    out3 = _emb_lookup(xT, tp)
    return jnp.transpose(out3, (2, 0, 1))


# speedup vs baseline: 1.0030x; 1.0030x over previous
import functools
import math

import jax
import jax.numpy as jnp
from jax import lax
from jax.experimental import pallas as pl
from jax.experimental.pallas import tpu as pltpu
from jax.experimental.pallas import tpu_sc as plsc

HIDDEN = 64
SCALE = math.sqrt(HIDDEN)

NC = 2
NS = 16
NW = NC * NS

NTOK = 16384
SEQ = 50
VOCAB = 1000000
HALF = VOCAB // 2
IBS = 128
NIB = NTOK // IBS
ITEMS = SEQ * NIB
IPW = ITEMS // NW

_mesh = plsc.VectorSubcoreMesh(core_axis_name="c", subcore_axis_name="s")


@functools.partial(
    pl.kernel,
    mesh=_mesh,
    out_type=jax.ShapeDtypeStruct((SEQ, HIDDEN, NTOK), jnp.float32),
    scratch_types=[
        pltpu.VMEM((2, 8, IBS), jnp.int32),
        pltpu.VMEM((2, IBS), jnp.int32),
        pltpu.VMEM((2, IBS), jnp.int32),
        pltpu.VMEM((2, IBS, 128), jnp.float32),
        pltpu.VMEM((2, HIDDEN, IBS), jnp.float32),
        pltpu.SemaphoreType.DMA,
        pltpu.SemaphoreType.DMA,
        pltpu.SemaphoreType.DMA,
    ],
    compiler_params=pltpu.CompilerParams(
        use_tc_tiling_on_sc=True, needs_layout_passes=False
    ),
)
def _emb_lookup(
    xT_hbm, tp_hbm, out_hbm, idx_v, pidx_v, par_v, rows_v, outt_v,
    isem, gsem, ssem,
):
    wid = lax.axis_index("s") * NC + lax.axis_index("c")
    iota16 = lax.iota(jnp.int32, 16)
    diag = [(iota16 + k) & 15 for k in range(16)]
    base0 = wid * IPW

    def coords(m):
        item = base0 + m
        j = item // NIB
        ib = item - j * NIB
        return j, ib

    def idx_copy(m):
        j, ib = coords(m)
        return pltpu.make_async_copy(
            xT_hbm.at[pl.ds((j // 8) * 8, 8), pl.ds(ib * IBS, IBS)],
            idx_v.at[m & 1],
            isem,
        )

    def gather_copy(m):
        return pltpu.make_async_copy(
            tp_hbm.at[pidx_v.at[m & 1]], rows_v.at[m & 1], gsem
        )

    def store_copy(m):
        j, ib = coords(m)
        return pltpu.make_async_copy(
            outt_v.at[m & 1], out_hbm.at[j, :, pl.ds(ib * IBS, IBS)], ssem
        )

    def prep(m):
        buf = m & 1
        j, _ = coords(m)
        jr = j - (j // 8) * 8

        def pidx_body(k, _):
            sl = pl.ds(k * 16, 16)
            iv = idx_v[buf, jr, sl]
            pidx_v[buf, sl] = lax.shift_right_logical(iv, 1)
            par_v[buf, sl] = (iv & 1) * HIDDEN
            return 0

        lax.fori_loop(0, IBS // 16, pidx_body, 0)
        gather_copy(m).start()

    idx_copy(0).start()
    idx_copy(0).wait()
    prep(0)
    idx_copy(1).start()

    def item_body(n, _):
        buf = n & 1

        @pl.when(n + 1 < IPW)
        def _prep_next():
            idx_copy(n + 1).wait()
            prep(n + 1)

        @pl.when(n + 2 < IPW)
        def _stage_next2():
            idx_copy(n + 2).start()

        gather_copy(n).wait()

        @pl.when(n >= 2)
        def _drain():
            store_copy(n - 2).wait()

        rows = rows_v.at[buf]
        outt = outt_v.at[buf]

        def grp_body(g, _):
            i0 = g * 16
            icol = i0 + iota16
            colbase = par_v[buf, pl.ds(i0, 16)]
            for hblk in range(0, HIDDEN, 16):
                cbh = colbase + hblk
                for k in range(16):
                    d = diag[k]
                    v = plsc.load_gather(rows, [icol, cbh + d])
                    plsc.store_scatter(outt, [hblk + d, icol], v * SCALE)
            return 0

        lax.fori_loop(0, IBS // 16, grp_body, 0)

        store_copy(n).start()
        return 0

    lax.fori_loop(0, IPW, item_body, 0)

    store_copy(IPW - 2).wait()
    store_copy(IPW - 1).wait()


def kernel(x, table):
    assert x.shape == (NTOK, SEQ) and table.shape == (VOCAB, HIDDEN)
    xT = x.astype(jnp.int32).T
    tp = table.reshape(HALF, 2 * HIDDEN)
    out3 = _emb_lookup(xT, tp)
    return jnp.transpose(out3, (2, 0, 1))
